# Initial kernel scaffold; baseline (speedup 1.0000x reference)
import jax
import jax.numpy as jnp
from jax.experimental import pallas as pl


def kernel(tokens, W):
    raise NotImplementedError("write your pallas kernel here")


# speedup vs baseline: 1.3995x; 1.3995x over previous
import functools
import math

import jax
import jax.numpy as jnp
from jax import lax
from jax.experimental import pallas as pl
from jax.experimental.pallas import tpu as pltpu
from jax.experimental.pallas import tpu_sc as plsc

VOCAB = 1_000_000
EMB = 32
B = 4096
L = 200
N = B * L

NC = 2
NS = 16
NW = NC * NS

IDX_MINOR = 128
ROWS_PER_W = N // NW
IROWS_PER_W = ROWS_PER_W // IDX_MINOR
CHUNK_IROWS = 8
CHUNK_TOK = CHUNK_IROWS * IDX_MINOR
N_CHUNKS = IROWS_PER_W // CHUNK_IROWS

SCALE = math.sqrt(float(EMB))

_mesh = plsc.VectorSubcoreMesh(
    core_axis_name="c", subcore_axis_name="s", num_cores=NC, num_subcores=NS
)


@functools.partial(
    pl.kernel,
    out_type=jax.ShapeDtypeStruct((N, EMB), jnp.float32),
    mesh=_mesh,
    scratch_types=[
        pltpu.VMEM((CHUNK_IROWS, IDX_MINOR), jnp.int32),
        pltpu.VMEM((CHUNK_TOK, EMB), jnp.float32),
        pltpu.SemaphoreType.DMA,
    ],
    compiler_params=pltpu.CompilerParams(use_tc_tiling_on_sc=False),
)
def _emb_lookup(idx_hbm, table_hbm, out_hbm, idx_v, rows_v, sem):
    wid = lax.axis_index("s") * NC + lax.axis_index("c")
    base_irow = wid * IROWS_PER_W

    def chunk_body(ci, carry):
        irow = base_irow + ci * CHUNK_IROWS
        tok_off = irow * IDX_MINOR
        pltpu.sync_copy(idx_hbm.at[pl.ds(irow, CHUNK_IROWS)], idx_v)
        for j in range(CHUNK_IROWS):
            pltpu.async_copy(
                table_hbm.at[idx_v.at[j]],
                rows_v.at[pl.ds(j * IDX_MINOR, IDX_MINOR)],
                sem,
            )
        for j in range(CHUNK_IROWS):
            pltpu.make_async_copy(
                table_hbm.at[idx_v.at[j]],
                rows_v.at[pl.ds(j * IDX_MINOR, IDX_MINOR)],
                sem,
            ).wait()

        def scale_body(r4, c):
            r = r4 * 4
            for u in range(4):
                rows_v[r + u, pl.ds(0, 16)] = rows_v[r + u, pl.ds(0, 16)] * SCALE
                rows_v[r + u, pl.ds(16, 16)] = rows_v[r + u, pl.ds(16, 16)] * SCALE
            return c

        lax.fori_loop(0, CHUNK_TOK // 4, scale_body, 0, unroll=2)
        pltpu.sync_copy(rows_v, out_hbm.at[pl.ds(tok_off, CHUNK_TOK)])
        return carry

    lax.fori_loop(0, N_CHUNKS, chunk_body, 0)


def kernel(tokens, W):
    idx = jnp.reshape(tokens.astype(jnp.int32), (N // IDX_MINOR, IDX_MINOR))
    out = _emb_lookup(idx, W)
    return jnp.reshape(out, (B, L, EMB))
